# Initial kernel scaffold; baseline (speedup 1.0000x reference)
import jax
import jax.numpy as jnp
from jax.experimental import pallas as pl


def kernel(support_features, query_features, support_labels, query_labels):
    raise NotImplementedError("write your pallas kernel here")


# speedup vs baseline: 1.9137x; 1.9137x over previous
import jax
import jax.numpy as jnp
from jax import lax
from jax.experimental import pallas as pl
from jax.experimental.pallas import tpu as pltpu
from jax.experimental.pallas import tpu_sc as plsc

NS, NQ, D, C = 100000, 4096, 64, 1000
PAD_CLASS = 1000
C_ACC = 1024
WPC = 80
ACC_W = C_ACC * WPC
NS_PAD = 102400
NCORES, NSUB = 2, 16
NW = NCORES * NSUB
ROWS_PER_W = NS_PAD // NW
CHUNK = 64
NCHUNKS = ROWS_PER_W // CHUNK
CWORDS = ACC_W // NW


def _acc_inner(sf_hbm, idx_hbm, zeros_hbm, out_ref, rows_v, idx_v, sem,
               acc_v):
    cid = lax.axis_index("c")
    sid = lax.axis_index("s")
    wid = cid * NSUB + sid
    base_row = wid * ROWS_PER_W
    ones16 = jnp.ones((16,), jnp.float32)
    pltpu.sync_copy(zeros_hbm, acc_v)

    def chunk_body(ic, carry):
        sl = pl.ds(base_row + ic * CHUNK, CHUNK)
        pltpu.sync_copy(sf_hbm.at[sl], rows_v)
        pltpu.sync_copy(idx_hbm.at[sl], idx_v)

        def grp_body(g, c2):
            cvec = idx_v[pl.ds(g * 16, 16)]
            for l in range(16):
                base = cvec[l] * WPC
                r = g * 16 + l
                for j in range(D // 16):
                    o = pl.ds(base + j * 16, 16)
                    acc_v[o] = acc_v[o] + rows_v[r, pl.ds(j * 16, 16)]
                oc = pl.ds(base + D, 16)
                acc_v[oc] = acc_v[oc] + ones16
            return c2

        lax.fori_loop(0, CHUNK // 16, grp_body, 0)
        return carry

    lax.fori_loop(0, NCHUNKS, chunk_body, 0)
    pltpu.sync_copy(acc_v, out_ref.at[wid])


def _acc_body(sf_hbm, idx_hbm, zeros_hbm, out_ref, rows_v, idx_v, sem):
    pl.run_scoped(
        lambda acc_v: _acc_inner(sf_hbm, idx_hbm, zeros_hbm, out_ref,
                                 rows_v, idx_v, sem, acc_v),
        pltpu.VMEM((ACC_W,), jnp.float32))


def _combine_inner(in_ref, out_ref, sem, buf_v, acc_v):
    cid = lax.axis_index("c")
    sid = lax.axis_index("s")
    wid = cid * NSUB + sid
    sl = pl.ds(wid * CWORDS, CWORDS)
    zero16 = jnp.zeros((16,), jnp.float32)

    def zrow(r, carry):
        acc_v[pl.ds(r * 16, 16)] = zero16
        return carry

    lax.fori_loop(0, CWORDS // 16, zrow, 0)
    for grp in range(2):
        copies = [
            pltpu.async_copy(in_ref.at[grp * 16 + w].at[sl], buf_v.at[w],
                             sem)
            for w in range(16)
        ]
        for cp in copies:
            cp.wait()

        def add_w(w, carry):
            def add_r(r, c2):
                o = pl.ds(r * 16, 16)
                acc_v[o] = acc_v[o] + buf_v[w, o]
                return c2

            lax.fori_loop(0, CWORDS // 16, add_r, 0)
            return carry

        lax.fori_loop(0, 16, add_w, 0)
    pltpu.sync_copy(acc_v, out_ref.at[sl])


def _combine_body(in_ref, out_ref, sem):
    pl.run_scoped(
        lambda buf_v, acc_v: _combine_inner(in_ref, out_ref, sem,
                                            buf_v, acc_v),
        pltpu.VMEM((16, CWORDS), jnp.float32),
        pltpu.VMEM((CWORDS,), jnp.float32))


_MESH = plsc.VectorSubcoreMesh(core_axis_name="c", subcore_axis_name="s")


def _segment_sums(sf, idx1d, zeros1d):
    acc_call = pl.kernel(
        _acc_body,
        out_type=pltpu.HBM((NW, ACC_W), jnp.float32),
        mesh=_MESH,
        scratch_types=[
            pltpu.VMEM((CHUNK, D), jnp.float32),
            pltpu.VMEM((CHUNK,), jnp.int32),
            pltpu.SemaphoreType.DMA,
        ],
    )
    partials = acc_call(sf, idx1d, zeros1d)

    combine_call = pl.kernel(
        _combine_body,
        out_type=pltpu.HBM((ACC_W,), jnp.float32),
        mesh=_MESH,
        scratch_types=[pltpu.SemaphoreType.DMA],
    )
    return combine_call(partials)


def _proto_body(a_ref, pn_ref):
    a = a_ref[...]
    sums = a[:, :D]
    cnt = jnp.sum(a[:, D:], axis=1, keepdims=True) * (1.0 / 16.0)
    protos = sums / jnp.maximum(cnt, 1.0)
    pn_ref[...] = protos / jnp.maximum(
        jnp.sqrt(jnp.sum(protos * protos, axis=1, keepdims=True)), 1e-8)


def _prototypes(combined):
    return pl.pallas_call(
        _proto_body,
        out_shape=jax.ShapeDtypeStruct((C_ACC, D), jnp.float32),
    )(combined.reshape(C_ACC, WPC))


QBLK = 1024


def _tc_body(pn_ref, q_ref, qcls_ref, acc_ref):
    i = pl.program_id(0)
    pn = pn_ref[...]
    q = q_ref[...]
    qn = q / jnp.maximum(
        jnp.sqrt(jnp.sum(q * q, axis=1, keepdims=True)), 1e-8)
    sim = lax.dot_general(qn, pn, (((1,), (1,)), ((), ())),
                          preferred_element_type=jnp.float32)
    col = lax.broadcasted_iota(jnp.int32, sim.shape, 1)
    sim = jnp.where(col < C, sim, -jnp.inf)
    rowmax = jnp.max(sim, axis=1, keepdims=True)
    pred = jnp.min(jnp.where(sim == rowmax, col, 2 ** 30), axis=1,
                   keepdims=True)
    frac = jnp.sum((pred == qcls_ref[...]).astype(jnp.float32)) * (1.0 / NQ)

    @pl.when(i == 0)
    def _():
        acc_ref[0, 0] = 0.0

    acc_ref[0, 0] += frac


def _classify(pn, qf, qcls2d):
    return pl.pallas_call(
        _tc_body,
        grid=(NQ // QBLK,),
        in_specs=[
            pl.BlockSpec((C_ACC, D), lambda i: (0, 0)),
            pl.BlockSpec((QBLK, D), lambda i: (i, 0)),
            pl.BlockSpec((QBLK, 1), lambda i: (i, 0)),
        ],
        out_specs=pl.BlockSpec(memory_space=pltpu.SMEM),
        out_shape=jax.ShapeDtypeStruct((1, 1), jnp.float32),
    )(pn, qf, qcls2d)


def kernel(support_features, query_features, support_labels, query_labels):
    s_class = support_labels[:, 0].astype(jnp.int32)
    q_class = query_labels[:, 0].astype(jnp.int32)
    sf = jnp.pad(support_features, ((0, NS_PAD - NS), (0, 0)))
    idx1d = jnp.pad(s_class, (0, NS_PAD - NS), constant_values=PAD_CLASS)
    zeros1d = jnp.zeros((ACC_W,), jnp.float32)
    combined = _segment_sums(sf, idx1d, zeros1d)
    pn = _prototypes(combined)
    acc = _classify(pn, query_features, q_class.reshape(NQ, 1))
    return acc.reshape(())


# speedup vs baseline: 2.0381x; 1.0650x over previous
import jax
import jax.numpy as jnp
from jax import lax
from jax.experimental import pallas as pl
from jax.experimental.pallas import tpu as pltpu
from jax.experimental.pallas import tpu_sc as plsc

NS, NQ, D, C = 100000, 4096, 64, 1000
PAD_CLASS = 1000
C_ACC = 1024
WPC = 80
ACC_W = C_ACC * WPC
NS_PAD = 102400
NCORES, NSUB = 2, 16
NW = NCORES * NSUB
ROWS_PER_W = NS_PAD // NW
CHUNK = 64
NCHUNKS = ROWS_PER_W // CHUNK
CWORDS = ACC_W // NW


def _acc_inner(sf_hbm, idx_hbm, zeros_hbm, out_ref, rows_v, idx_v, sem,
               acc_v):
    cid = lax.axis_index("c")
    sid = lax.axis_index("s")
    wid = cid * NSUB + sid
    base_row = wid * ROWS_PER_W
    ones16 = jnp.ones((16,), jnp.float32)
    pltpu.sync_copy(zeros_hbm, acc_v)

    def chunk_body(ic, carry):
        sl = pl.ds(base_row + ic * CHUNK, CHUNK)
        pltpu.sync_copy(sf_hbm.at[sl], rows_v)
        pltpu.sync_copy(idx_hbm.at[sl], idx_v)

        def grp_body(g, c2):
            cvec = idx_v[pl.ds(g * 16, 16)] * WPC
            for l in range(16):
                base = cvec[l]
                r = g * 16 + l
                for j in range(D // 16):
                    plsc.addupdate(acc_v.at[pl.ds(base + j * 16, 16)],
                                   rows_v[r, pl.ds(j * 16, 16)])
                plsc.addupdate(acc_v.at[pl.ds(base + D, 16)], ones16)
            return c2

        lax.fori_loop(0, CHUNK // 16, grp_body, 0)
        return carry

    lax.fori_loop(0, NCHUNKS, chunk_body, 0)
    pltpu.sync_copy(acc_v, out_ref.at[wid])


def _acc_body(sf_hbm, idx_hbm, zeros_hbm, out_ref, rows_v, idx_v, sem):
    pl.run_scoped(
        lambda acc_v: _acc_inner(sf_hbm, idx_hbm, zeros_hbm, out_ref,
                                 rows_v, idx_v, sem, acc_v),
        pltpu.VMEM((ACC_W,), jnp.float32))


def _combine_inner(in_ref, out_ref, sem, buf_v, acc_v):
    cid = lax.axis_index("c")
    sid = lax.axis_index("s")
    wid = cid * NSUB + sid
    sl = pl.ds(wid * CWORDS, CWORDS)
    zero16 = jnp.zeros((16,), jnp.float32)

    def zrow(r, carry):
        acc_v[pl.ds(r * 16, 16)] = zero16
        return carry

    lax.fori_loop(0, CWORDS // 16, zrow, 0)
    for grp in range(2):
        copies = [
            pltpu.async_copy(in_ref.at[grp * 16 + w].at[sl], buf_v.at[w],
                             sem)
            for w in range(16)
        ]
        for cp in copies:
            cp.wait()

        def add_w(w, carry):
            def add_r(r, c2):
                o = pl.ds(r * 16, 16)
                acc_v[o] = acc_v[o] + buf_v[w, o]
                return c2

            lax.fori_loop(0, CWORDS // 16, add_r, 0)
            return carry

        lax.fori_loop(0, 16, add_w, 0)
    pltpu.sync_copy(acc_v, out_ref.at[sl])


def _combine_body(in_ref, out_ref, sem):
    pl.run_scoped(
        lambda buf_v, acc_v: _combine_inner(in_ref, out_ref, sem,
                                            buf_v, acc_v),
        pltpu.VMEM((16, CWORDS), jnp.float32),
        pltpu.VMEM((CWORDS,), jnp.float32))


_MESH = plsc.VectorSubcoreMesh(core_axis_name="c", subcore_axis_name="s")


def _segment_sums(sf, idx1d, zeros1d):
    acc_call = pl.kernel(
        _acc_body,
        out_type=pltpu.HBM((NW, ACC_W), jnp.float32),
        mesh=_MESH,
        scratch_types=[
            pltpu.VMEM((CHUNK, D), jnp.float32),
            pltpu.VMEM((CHUNK,), jnp.int32),
            pltpu.SemaphoreType.DMA,
        ],
    )
    partials = acc_call(sf, idx1d, zeros1d)

    combine_call = pl.kernel(
        _combine_body,
        out_type=pltpu.HBM((ACC_W,), jnp.float32),
        mesh=_MESH,
        scratch_types=[pltpu.SemaphoreType.DMA],
    )
    return combine_call(partials)


def _proto_body(a_ref, pn_ref):
    a = a_ref[...]
    sums = a[:, :D]
    cnt = jnp.sum(a[:, D:], axis=1, keepdims=True) * (1.0 / 16.0)
    protos = sums / jnp.maximum(cnt, 1.0)
    pn_ref[...] = protos / jnp.maximum(
        jnp.sqrt(jnp.sum(protos * protos, axis=1, keepdims=True)), 1e-8)


def _prototypes(combined):
    return pl.pallas_call(
        _proto_body,
        out_shape=jax.ShapeDtypeStruct((C_ACC, D), jnp.float32),
    )(combined.reshape(C_ACC, WPC))


QBLK = 1024


def _tc_body(pn_ref, q_ref, qcls_ref, acc_ref):
    i = pl.program_id(0)
    pn = pn_ref[...]
    q = q_ref[...]
    qn = q / jnp.maximum(
        jnp.sqrt(jnp.sum(q * q, axis=1, keepdims=True)), 1e-8)
    sim = lax.dot_general(qn, pn, (((1,), (1,)), ((), ())),
                          preferred_element_type=jnp.float32)
    col = lax.broadcasted_iota(jnp.int32, sim.shape, 1)
    sim = jnp.where(col < C, sim, -jnp.inf)
    rowmax = jnp.max(sim, axis=1, keepdims=True)
    pred = jnp.min(jnp.where(sim == rowmax, col, 2 ** 30), axis=1,
                   keepdims=True)
    frac = jnp.sum((pred == qcls_ref[...]).astype(jnp.float32)) * (1.0 / NQ)

    @pl.when(i == 0)
    def _():
        acc_ref[0, 0] = 0.0

    acc_ref[0, 0] += frac


def _classify(pn, qf, qcls2d):
    return pl.pallas_call(
        _tc_body,
        grid=(NQ // QBLK,),
        in_specs=[
            pl.BlockSpec((C_ACC, D), lambda i: (0, 0)),
            pl.BlockSpec((QBLK, D), lambda i: (i, 0)),
            pl.BlockSpec((QBLK, 1), lambda i: (i, 0)),
        ],
        out_specs=pl.BlockSpec(memory_space=pltpu.SMEM),
        out_shape=jax.ShapeDtypeStruct((1, 1), jnp.float32),
    )(pn, qf, qcls2d)


def kernel(support_features, query_features, support_labels, query_labels):
    s_class = support_labels[:, 0].astype(jnp.int32)
    q_class = query_labels[:, 0].astype(jnp.int32)
    sf = jnp.pad(support_features, ((0, NS_PAD - NS), (0, 0)))
    idx1d = jnp.pad(s_class, (0, NS_PAD - NS), constant_values=PAD_CLASS)
    zeros1d = jnp.zeros((ACC_W,), jnp.float32)
    combined = _segment_sums(sf, idx1d, zeros1d)
    pn = _prototypes(combined)
    acc = _classify(pn, query_features, q_class.reshape(NQ, 1))
    return acc.reshape(())


# speedup vs baseline: 2.7364x; 1.3426x over previous
import jax
import jax.numpy as jnp
from jax import lax
from jax.experimental import pallas as pl
from jax.experimental.pallas import tpu as pltpu
from jax.experimental.pallas import tpu_sc as plsc

NS, NQ, D, C = 100000, 4096, 64, 1000
PAD_CLASS = 1000
C_ACC = 1024
WPC = 80
ACC_W = C_ACC * WPC
NS_PAD = 102400
NCORES, NSUB = 2, 16
NW = NCORES * NSUB
ROWS_PER_W = NS_PAD // NW
CHUNK = 64
NCHUNKS = ROWS_PER_W // CHUNK
CWORDS = ACC_W // NW


def _acc_inner(sf_hbm, idx_hbm, zeros_hbm, out_ref, rows_a, rows_b, idx_v,
               sem_a, sem_b, acc_v):
    cid = lax.axis_index("c")
    sid = lax.axis_index("s")
    wid = cid * NSUB + sid
    base_row = wid * ROWS_PER_W
    ones16 = jnp.ones((16,), jnp.float32)
    pltpu.sync_copy(idx_hbm.at[pl.ds(base_row, ROWS_PER_W)], idx_v)
    pltpu.sync_copy(zeros_hbm, acc_v)

    def compute_chunk(ic, rows_v):
        def grp_body(g, c2):
            cvec = idx_v[pl.ds(ic * CHUNK + g * 16, 16)] * WPC
            for l in range(16):
                base = cvec[l]
                r = g * 16 + l
                for j in range(D // 16):
                    plsc.addupdate(acc_v.at[pl.ds(base + j * 16, 16)],
                                   rows_v[r, pl.ds(j * 16, 16)])
                plsc.addupdate(acc_v.at[pl.ds(base + D, 16)], ones16)
            return c2

        lax.fori_loop(0, CHUNK // 16, grp_body, 0)

    def start(ic, buf, sem):
        pltpu.async_copy(sf_hbm.at[pl.ds(base_row + ic * CHUNK, CHUNK)],
                         buf, sem)

    def drain(buf, sem):
        pltpu.make_async_copy(sf_hbm.at[pl.ds(base_row, CHUNK)], buf,
                              sem).wait()

    start(0, rows_a, sem_a)

    def pair_body(ip, carry):
        drain(rows_a, sem_a)
        start(2 * ip + 1, rows_b, sem_b)
        compute_chunk(2 * ip, rows_a)
        drain(rows_b, sem_b)
        start(2 * ip + 2, rows_a, sem_a)
        compute_chunk(2 * ip + 1, rows_b)
        return carry

    lax.fori_loop(0, NCHUNKS // 2, pair_body, 0)
    drain(rows_a, sem_a)
    compute_chunk(NCHUNKS - 1, rows_a)
    pltpu.sync_copy(acc_v, out_ref.at[wid])


def _acc_body(sf_hbm, idx_hbm, zeros_hbm, out_ref, rows_a, rows_b, idx_v,
              sem_a, sem_b):
    pl.run_scoped(
        lambda acc_v: _acc_inner(sf_hbm, idx_hbm, zeros_hbm, out_ref,
                                 rows_a, rows_b, idx_v, sem_a, sem_b,
                                 acc_v),
        pltpu.VMEM((ACC_W,), jnp.float32))


def _combine_inner(in_ref, out_ref, sem, buf_v, acc_v):
    cid = lax.axis_index("c")
    sid = lax.axis_index("s")
    wid = cid * NSUB + sid
    sl = pl.ds(wid * CWORDS, CWORDS)
    zero16 = jnp.zeros((16,), jnp.float32)

    def zrow(r, carry):
        acc_v[pl.ds(r * 16, 16)] = zero16
        return carry

    lax.fori_loop(0, CWORDS // 16, zrow, 0)
    for grp in range(2):
        copies = [
            pltpu.async_copy(in_ref.at[grp * 16 + w].at[sl], buf_v.at[w],
                             sem)
            for w in range(16)
        ]
        for cp in copies:
            cp.wait()

        def add_w(w, carry):
            def add_r(r, c2):
                o = pl.ds(r * 16, 16)
                acc_v[o] = acc_v[o] + buf_v[w, o]
                return c2

            lax.fori_loop(0, CWORDS // 16, add_r, 0)
            return carry

        lax.fori_loop(0, 16, add_w, 0)
    pltpu.sync_copy(acc_v, out_ref.at[sl])


def _combine_body(in_ref, out_ref, sem):
    pl.run_scoped(
        lambda buf_v, acc_v: _combine_inner(in_ref, out_ref, sem,
                                            buf_v, acc_v),
        pltpu.VMEM((16, CWORDS), jnp.float32),
        pltpu.VMEM((CWORDS,), jnp.float32))


_MESH = plsc.VectorSubcoreMesh(core_axis_name="c", subcore_axis_name="s")


def _segment_sums(sf, idx1d, zeros1d):
    acc_call = pl.kernel(
        _acc_body,
        out_type=pltpu.HBM((NW, ACC_W), jnp.float32),
        mesh=_MESH,
        scratch_types=[
            pltpu.VMEM((CHUNK, D), jnp.float32),
            pltpu.VMEM((CHUNK, D), jnp.float32),
            pltpu.VMEM((ROWS_PER_W,), jnp.int32),
            pltpu.SemaphoreType.DMA,
            pltpu.SemaphoreType.DMA,
        ],
    )
    partials = acc_call(sf, idx1d, zeros1d)

    combine_call = pl.kernel(
        _combine_body,
        out_type=pltpu.HBM((ACC_W,), jnp.float32),
        mesh=_MESH,
        scratch_types=[pltpu.SemaphoreType.DMA],
    )
    return combine_call(partials)


def _proto_body(a_ref, pn_ref):
    a = a_ref[...]
    sums = a[:, :D]
    cnt = jnp.sum(a[:, D:], axis=1, keepdims=True) * (1.0 / 16.0)
    protos = sums / jnp.maximum(cnt, 1.0)
    pn_ref[...] = protos / jnp.maximum(
        jnp.sqrt(jnp.sum(protos * protos, axis=1, keepdims=True)), 1e-8)


def _prototypes(combined):
    return pl.pallas_call(
        _proto_body,
        out_shape=jax.ShapeDtypeStruct((C_ACC, D), jnp.float32),
    )(combined.reshape(C_ACC, WPC))


QBLK = 1024


def _tc_body(pn_ref, q_ref, qcls_ref, acc_ref):
    i = pl.program_id(0)
    pn = pn_ref[...]
    q = q_ref[...]
    qn = q / jnp.maximum(
        jnp.sqrt(jnp.sum(q * q, axis=1, keepdims=True)), 1e-8)
    sim = lax.dot_general(qn, pn, (((1,), (1,)), ((), ())),
                          preferred_element_type=jnp.float32)
    col = lax.broadcasted_iota(jnp.int32, sim.shape, 1)
    sim = jnp.where(col < C, sim, -jnp.inf)
    rowmax = jnp.max(sim, axis=1, keepdims=True)
    pred = jnp.min(jnp.where(sim == rowmax, col, 2 ** 30), axis=1,
                   keepdims=True)
    frac = jnp.sum((pred == qcls_ref[...]).astype(jnp.float32)) * (1.0 / NQ)

    @pl.when(i == 0)
    def _():
        acc_ref[0, 0] = 0.0

    acc_ref[0, 0] += frac


def _classify(pn, qf, qcls2d):
    return pl.pallas_call(
        _tc_body,
        grid=(NQ // QBLK,),
        in_specs=[
            pl.BlockSpec((C_ACC, D), lambda i: (0, 0)),
            pl.BlockSpec((QBLK, D), lambda i: (i, 0)),
            pl.BlockSpec((QBLK, 1), lambda i: (i, 0)),
        ],
        out_specs=pl.BlockSpec(memory_space=pltpu.SMEM),
        out_shape=jax.ShapeDtypeStruct((1, 1), jnp.float32),
    )(pn, qf, qcls2d)


def kernel(support_features, query_features, support_labels, query_labels):
    s_class = support_labels[:, 0].astype(jnp.int32)
    q_class = query_labels[:, 0].astype(jnp.int32)
    sf = jnp.pad(support_features, ((0, NS_PAD - NS), (0, 0)))
    idx1d = jnp.pad(s_class, (0, NS_PAD - NS), constant_values=PAD_CLASS)
    zeros1d = jnp.zeros((ACC_W,), jnp.float32)
    combined = _segment_sums(sf, idx1d, zeros1d)
    pn = _prototypes(combined)
    acc = _classify(pn, query_features, q_class.reshape(NQ, 1))
    return acc.reshape(())
